# Initial kernel scaffold; baseline (speedup 1.0000x reference)
import jax
import jax.numpy as jnp
from jax.experimental import pallas as pl


def kernel(input_ids, val):
    raise NotImplementedError("write your pallas kernel here")


# speedup vs baseline: 1.8778x; 1.8778x over previous
import jax
import jax.numpy as jnp
from jax.experimental import pallas as pl

VOCAB = 100000
ROWS_BLK = 64
V_BLK = 12800


def _onehot_block(pred_ref, val_ref, out_ref):
    v_block = pl.program_id(1)
    pred = pred_ref[:, 0]
    val = val_ref[:, 0]
    iota = jax.lax.broadcasted_iota(jnp.int32, (ROWS_BLK, V_BLK), 1)
    iota = iota + v_block * V_BLK
    out_ref[...] = jnp.where(
        iota == pred[:, None], 50.0 * val[:, None],
        jnp.float32(-50.0))


def kernel(input_ids, val):
    B, S = input_ids.shape
    rows = B * S
    pred = input_ids.reshape(rows, 1)
    val2 = val.reshape(rows, 1)
    n_row_blocks = rows // ROWS_BLK
    n_v_blocks = (VOCAB + V_BLK - 1) // V_BLK
    out = pl.pallas_call(
        _onehot_block,
        grid=(n_row_blocks, n_v_blocks),
        in_specs=[
            pl.BlockSpec((ROWS_BLK, 1), lambda i, j: (i, 0)),
            pl.BlockSpec((ROWS_BLK, 1), lambda i, j: (i, 0)),
        ],
        out_specs=pl.BlockSpec((ROWS_BLK, V_BLK), lambda i, j: (i, j)),
        out_shape=jax.ShapeDtypeStruct((rows, VOCAB), jnp.float32),
    )(pred, val2)
    return out.reshape(B, S, VOCAB)


# speedup vs baseline: 2.0172x; 1.0743x over previous
import jax
import jax.numpy as jnp
from jax.experimental import pallas as pl

VOCAB = 100000
ROWS_BLK = 128
V_BLK = 12800


def _onehot_block(pred_ref, val_ref, out_ref):
    v_block = pl.program_id(1)
    pred = pred_ref[:, 0]
    val = val_ref[:, 0]
    iota = jax.lax.broadcasted_iota(jnp.int32, (ROWS_BLK, V_BLK), 1)
    iota = iota + v_block * V_BLK
    out_ref[...] = jnp.where(
        iota == pred[:, None], 50.0 * val[:, None],
        jnp.float32(-50.0))


def kernel(input_ids, val):
    B, S = input_ids.shape
    rows = B * S
    pred = input_ids.reshape(rows, 1)
    val2 = val.reshape(rows, 1)
    n_row_blocks = rows // ROWS_BLK
    n_v_blocks = (VOCAB + V_BLK - 1) // V_BLK
    out = pl.pallas_call(
        _onehot_block,
        grid=(n_row_blocks, n_v_blocks),
        in_specs=[
            pl.BlockSpec((ROWS_BLK, 1), lambda i, j: (i, 0)),
            pl.BlockSpec((ROWS_BLK, 1), lambda i, j: (i, 0)),
        ],
        out_specs=pl.BlockSpec((ROWS_BLK, V_BLK), lambda i, j: (i, j)),
        out_shape=jax.ShapeDtypeStruct((rows, VOCAB), jnp.float32),
    )(pred, val2)
    return out.reshape(B, S, VOCAB)
